# Initial kernel scaffold; baseline (speedup 1.0000x reference)
import jax
import jax.numpy as jnp
from jax.experimental import pallas as pl


def kernel(nodes, edge_index, edge_attr, valid, r, fx, loc, earray, params):
    raise NotImplementedError("write your pallas kernel here")


# speedup vs baseline: 1.8319x; 1.8319x over previous
import functools

import jax
import jax.numpy as jnp
from jax.experimental import pallas as pl


def _gelu(x):
    return 0.5 * x * (1.0 + jax.lax.erf(x * 0.7071067811865476))


def _edge_logits_body(ea_ref, w1_ref, b1_ref, w2_ref, b2_ref, w3_ref, b3_ref,
                      out_ref):
    x = ea_ref[:, 0:3]
    h = _gelu(jnp.dot(x, w1_ref[...], preferred_element_type=jnp.float32)
              + b1_ref[...])
    h = _gelu(jnp.dot(h, w2_ref[...], preferred_element_type=jnp.float32)
              + b2_ref[...])
    out_ref[...] = (jnp.dot(h, w3_ref[...], preferred_element_type=jnp.float32)
                    + b3_ref[...])


def _edge_logits(edge_attr, p):
    E = edge_attr.shape[0]
    BLK = 8000
    grid = (E // BLK,)
    w1 = p['le_w1'].T
    b1 = p['le_b1'].reshape(1, 16)
    w2 = p['le_w2'].T
    b2 = p['le_b2'].reshape(1, 16)
    w3 = p['le_w3'].T
    b3 = p['le_b3'].reshape(1, 1)
    full = lambda a: pl.BlockSpec(a.shape, lambda i: (0,) * a.ndim)
    out = pl.pallas_call(
        _edge_logits_body,
        grid=grid,
        in_specs=[
            pl.BlockSpec((BLK, edge_attr.shape[1]), lambda i: (i, 0)),
            full(w1), full(b1), full(w2), full(b2), full(w3), full(b3),
        ],
        out_specs=pl.BlockSpec((BLK, 1), lambda i: (i, 0)),
        out_shape=jax.ShapeDtypeStruct((E, 1), jnp.float32),
    )(edge_attr, w1, b1, w2, b2, w3, b3)
    return out[:, 0]


def _rain_body(x_ref, w1_ref, b1_ref, w2_ref, b2_ref, out_ref):
    x = x_ref[...]
    h = _gelu(x * w1_ref[...] + b1_ref[...])
    out_ref[...] = (jnp.dot(h, w2_ref[...], preferred_element_type=jnp.float32)
                    + b2_ref[...])


def _rain_mlp(r0, p):
    N = r0.shape[0]
    x = r0.reshape(N * 8, 1)
    BLK = 8000
    grid = ((N * 8) // BLK,)
    w1 = p['rain_w1'].reshape(1, 8)
    b1 = p['rain_b1'].reshape(1, 8)
    w2 = p['rain_w2'].T
    b2 = p['rain_b2'].reshape(1, 8)
    full = lambda a: pl.BlockSpec(a.shape, lambda i: (0,) * a.ndim)
    out = pl.pallas_call(
        _rain_body,
        grid=grid,
        in_specs=[
            pl.BlockSpec((BLK, 1), lambda i: (i, 0)),
            full(w1), full(b1), full(w2), full(b2),
        ],
        out_specs=pl.BlockSpec((BLK, 8), lambda i: (i, 0)),
        out_shape=jax.ShapeDtypeStruct((N * 8, 8), jnp.float32),
    )(x, w1, b1, w2, b2)
    return out.reshape(N, 8, 8)


def _gru_head_body(x_ref, wih_ref, bih_ref, whh_ref, bhh_ref, pw_ref, pb_ref,
                   out_ref):
    B = x_ref.shape[0]
    h = jnp.zeros((B, 8), jnp.float32)
    cols = []
    for t in range(8):
        xt = x_ref[:, 8 * t:8 * t + 8]
        gi = jnp.dot(xt, wih_ref[...], preferred_element_type=jnp.float32) \
            + bih_ref[...]
        gh = jnp.dot(h, whh_ref[...], preferred_element_type=jnp.float32) \
            + bhh_ref[...]
        rg = jax.nn.sigmoid(gi[:, 0:8] + gh[:, 0:8])
        zg = jax.nn.sigmoid(gi[:, 8:16] + gh[:, 8:16])
        ng = jnp.tanh(gi[:, 16:24] + rg * gh[:, 16:24])
        h = (1.0 - zg) * ng + zg * h
        cols.append(jnp.dot(_gelu(h), pw_ref[...],
                            preferred_element_type=jnp.float32) + pb_ref[...])
    out_ref[...] = jnp.concatenate(cols, axis=1)


def _gru_head(x, p):
    B = x.shape[0]
    x2 = x.reshape(B, 64)
    BLK = 1040
    grid = (B // BLK,)
    wih = p['gru_wih'].T
    bih = p['gru_bih'].reshape(1, 24)
    whh = p['gru_whh'].T
    bhh = p['gru_bhh'].reshape(1, 24)
    pw = p['post_w'].T
    pb = p['post_b'].reshape(1, 1)
    full = lambda a: pl.BlockSpec(a.shape, lambda i: (0,) * a.ndim)
    return pl.pallas_call(
        _gru_head_body,
        grid=grid,
        in_specs=[pl.BlockSpec((BLK, 64), lambda i: (i, 0)),
                  full(wih), full(bih), full(whh), full(bhh),
                  full(pw), full(pb)],
        out_specs=pl.BlockSpec((BLK, 8), lambda i: (i, 0)),
        out_shape=jax.ShapeDtypeStruct((B, 8), jnp.float32),
    )(x2, wih, bih, whh, bhh, pw, pb)


def _combine_body(xs_ref, res_ref, pred0_ref, w11_ref, b11_ref, w12_ref,
                  b12_ref, w21_ref, b21_ref, w22_ref, b22_ref, ww_ref, wb_ref,
                  out_ref):
    xs = xs_ref[...]
    h1 = _gelu(jnp.dot(xs, w11_ref[...], preferred_element_type=jnp.float32)
               + b11_ref[...])
    h1 = jnp.dot(h1, w12_ref[...], preferred_element_type=jnp.float32) \
        + b12_ref[...]
    h2 = _gelu(jnp.dot(h1, w21_ref[...], preferred_element_type=jnp.float32)
               + b21_ref[...])
    feats = jnp.dot(h2, w22_ref[...], preferred_element_type=jnp.float32) \
        + b22_ref[...]
    wt = jax.nn.relu(jnp.dot(feats, ww_ref[...],
                             preferred_element_type=jnp.float32) + wb_ref[...])
    alpha = wt / (jnp.sum(wt, axis=0, keepdims=True) + 1e-08)
    res = res_ref[...]
    colmask = jax.lax.broadcasted_iota(jnp.int32, res.shape, 1) != 0
    res = jnp.where(colmask, res, 0.0)
    pr = jnp.sum(res * alpha, axis=0, keepdims=True)
    out_ref[...] = pred0_ref[...] + pr


def _combine(xs, res, pred0, p):
    full = lambda a: pl.BlockSpec(a.shape, lambda *_: (0,) * a.ndim)
    args = [xs, res, pred0,
            p['m1_w1'].T, p['m1_b1'].reshape(1, 12),
            p['m1_w2'].T, p['m1_b2'].reshape(1, 8),
            p['m2_w1'].T, p['m2_b1'].reshape(1, 8),
            p['m2_w2'].T, p['m2_b2'].reshape(1, 8),
            p['wt_w'].T, p['wt_b'].reshape(1, 1)]
    return pl.pallas_call(
        _combine_body,
        in_specs=[full(a) for a in args],
        out_specs=pl.BlockSpec((1, 8), lambda *_: (0, 0)),
        out_shape=jax.ShapeDtypeStruct((1, 8), jnp.float32),
    )(*args)


def _hops_to_target0(src, dst, n):
    hop0 = jnp.full((n,), -1, dtype=jnp.int32).at[0].set(0)

    def cond(carry):
        hop, h = carry
        m = (hop[dst] == h) & (hop[src] == -1)
        return m.any()

    def body(carry):
        hop, h = carry
        m = (hop[dst] == h) & (hop[src] == -1)
        reached = jax.ops.segment_max(m.astype(jnp.int32), src,
                                      num_segments=n) > 0
        return jnp.where(reached, h + 1, hop), h + 1

    hop, _ = jax.lax.while_loop(cond, body, (hop0, jnp.int32(0)))
    return hop


def kernel(nodes, edge_index, edge_attr, valid, r, fx, loc, earray, params):
    p = params
    n = nodes.shape[0]
    res_idx = earray.shape[1]

    v = valid[0][:, :, None]
    nodes_m = jax.lax.stop_gradient(nodes * v)
    nodes_flat = nodes_m.reshape(n, -1)
    earray2 = jax.lax.stop_gradient(earray[0])
    vb = v != 0
    valid_mean = vb[:, :, 0].astype(jnp.float32).mean(axis=-1)

    src = edge_index[0].astype(jnp.int32)
    dst = edge_index[1].astype(jnp.int32)

    rf = _rain_mlp(r[0], p)

    logits = _edge_logits(edge_attr[:, :3], p) if edge_attr.shape[1] == 3 \
        else _edge_logits(edge_attr, p)

    hops = _hops_to_target0(src, dst, n)
    hs, hd = hops[src], hops[dst]
    on_path = (hs >= 0) & (hd >= 0) & (hs == hd + 1)
    max_hop = jnp.max(hops)

    neg_inf = jnp.float32(-jnp.inf)
    w = jnp.where(on_path, logits * valid_mean[src], neg_inf)
    m = jax.ops.segment_max(w, dst, num_segments=n)
    e = jnp.where(on_path, jnp.exp(w - m[dst]), 0.0)
    s = jax.ops.segment_sum(e, dst, num_segments=n)
    alpha = jnp.where(on_path, e / (s[dst] + 1e-16), 0.0)

    elev = jnp.where(on_path, hd + 1, 0)

    def hop_cond(carry):
        _, h = carry
        return h >= 1

    def hop_body(carry):
        x_out, h = carry
        a = jnp.where(elev == h, alpha, 0.0)
        x_flat = x_out.reshape(n, -1)
        msg = x_flat[src] * a[:, None]
        agg = jax.ops.segment_sum(msg, dst, num_segments=n)
        return x_out + agg.reshape(x_out.shape), h - 1

    x_out, _ = jax.lax.while_loop(hop_cond, hop_body, (rf, max_hop))

    B1 = res_idx
    B2 = res_idx + 1
    Bt = B1 + B2
    B_pad = ((Bt + 127) // 128) * 128
    batch = jnp.concatenate([
        rf[:B1],
        x_out[:B2],
        jnp.zeros((B_pad - Bt, 8, 8), jnp.float32),
    ], axis=0)
    head = _gru_head(batch, p)
    pred_coarse = head[:B1][:, :, None]
    pred_part = head[B1:B1 + B2]

    res = nodes_flat[1:res_idx + 1] - pred_part[1:]
    adjusted = _combine(earray2[:res_idx][:, :19], res, pred_part[:1], p)

    original_valid = vb[:, :, 0]
    return (adjusted, pred_part[:res_idx], pred_coarse, original_valid)


# speedup vs baseline: 1.8729x; 1.0224x over previous
import functools

import jax
import jax.numpy as jnp
from jax.experimental import pallas as pl


def _gelu(x):
    return 0.5 * x * (1.0 + jax.lax.erf(x * 0.7071067811865476))


def _edge_logits_body(ea_ref, w1_ref, b1_ref, w2_ref, b2_ref, w3_ref, b3_ref,
                      out_ref):
    x = ea_ref[:, 0:3]
    h = _gelu(jnp.dot(x, w1_ref[...], preferred_element_type=jnp.float32)
              + b1_ref[...])
    h = _gelu(jnp.dot(h, w2_ref[...], preferred_element_type=jnp.float32)
              + b2_ref[...])
    out_ref[...] = (jnp.dot(h, w3_ref[...], preferred_element_type=jnp.float32)
                    + b3_ref[...])


def _edge_logits(edge_attr, p):
    E = edge_attr.shape[0]
    BLK = 8000
    grid = (E // BLK,)
    w1 = p['le_w1'].T
    b1 = p['le_b1'].reshape(1, 16)
    w2 = p['le_w2'].T
    b2 = p['le_b2'].reshape(1, 16)
    w3 = p['le_w3'].T
    b3 = p['le_b3'].reshape(1, 1)
    full = lambda a: pl.BlockSpec(a.shape, lambda i: (0,) * a.ndim)
    out = pl.pallas_call(
        _edge_logits_body,
        grid=grid,
        in_specs=[
            pl.BlockSpec((BLK, edge_attr.shape[1]), lambda i: (i, 0)),
            full(w1), full(b1), full(w2), full(b2), full(w3), full(b3),
        ],
        out_specs=pl.BlockSpec((BLK, 1), lambda i: (i, 0)),
        out_shape=jax.ShapeDtypeStruct((E, 1), jnp.float32),
    )(edge_attr, w1, b1, w2, b2, w3, b3)
    return out[:, 0]


def _rain_body(x_ref, w1_ref, b1_ref, w2_ref, b2_ref, out_ref):
    x = x_ref[...]
    h = _gelu(x * w1_ref[...] + b1_ref[...])
    out_ref[...] = (jnp.dot(h, w2_ref[...], preferred_element_type=jnp.float32)
                    + b2_ref[...])


def _rain_mlp(r0, p):
    N = r0.shape[0]
    x = r0.reshape(N * 8, 1)
    BLK = 8000
    grid = ((N * 8) // BLK,)
    w1 = p['rain_w1'].reshape(1, 8)
    b1 = p['rain_b1'].reshape(1, 8)
    w2 = p['rain_w2'].T
    b2 = p['rain_b2'].reshape(1, 8)
    full = lambda a: pl.BlockSpec(a.shape, lambda i: (0,) * a.ndim)
    out = pl.pallas_call(
        _rain_body,
        grid=grid,
        in_specs=[
            pl.BlockSpec((BLK, 1), lambda i: (i, 0)),
            full(w1), full(b1), full(w2), full(b2),
        ],
        out_specs=pl.BlockSpec((BLK, 8), lambda i: (i, 0)),
        out_shape=jax.ShapeDtypeStruct((N * 8, 8), jnp.float32),
    )(x, w1, b1, w2, b2)
    return out.reshape(N, 8, 8)


def _gru_head_body(x_ref, wih_ref, bih_ref, whh_ref, bhh_ref, pw_ref, pb_ref,
                   out_ref):
    B = x_ref.shape[0]
    h = jnp.zeros((B, 8), jnp.float32)
    cols = []
    for t in range(8):
        xt = x_ref[:, 8 * t:8 * t + 8]
        gi = jnp.dot(xt, wih_ref[...], preferred_element_type=jnp.float32) \
            + bih_ref[...]
        gh = jnp.dot(h, whh_ref[...], preferred_element_type=jnp.float32) \
            + bhh_ref[...]
        rg = jax.nn.sigmoid(gi[:, 0:8] + gh[:, 0:8])
        zg = jax.nn.sigmoid(gi[:, 8:16] + gh[:, 8:16])
        ng = jnp.tanh(gi[:, 16:24] + rg * gh[:, 16:24])
        h = (1.0 - zg) * ng + zg * h
        cols.append(jnp.dot(_gelu(h), pw_ref[...],
                            preferred_element_type=jnp.float32) + pb_ref[...])
    out_ref[...] = jnp.concatenate(cols, axis=1)


def _gru_head(x, p):
    B = x.shape[0]
    x2 = x.reshape(B, 64)
    BLK = 1040
    grid = (B // BLK,)
    wih = p['gru_wih'].T
    bih = p['gru_bih'].reshape(1, 24)
    whh = p['gru_whh'].T
    bhh = p['gru_bhh'].reshape(1, 24)
    pw = p['post_w'].T
    pb = p['post_b'].reshape(1, 1)
    full = lambda a: pl.BlockSpec(a.shape, lambda i: (0,) * a.ndim)
    return pl.pallas_call(
        _gru_head_body,
        grid=grid,
        in_specs=[pl.BlockSpec((BLK, 64), lambda i: (i, 0)),
                  full(wih), full(bih), full(whh), full(bhh),
                  full(pw), full(pb)],
        out_specs=pl.BlockSpec((BLK, 8), lambda i: (i, 0)),
        out_shape=jax.ShapeDtypeStruct((B, 8), jnp.float32),
    )(x2, wih, bih, whh, bhh, pw, pb)


def _combine_body(xs_ref, res_ref, pred0_ref, w11_ref, b11_ref, w12_ref,
                  b12_ref, w21_ref, b21_ref, w22_ref, b22_ref, ww_ref, wb_ref,
                  out_ref):
    xs = xs_ref[...]
    h1 = _gelu(jnp.dot(xs, w11_ref[...], preferred_element_type=jnp.float32)
               + b11_ref[...])
    h1 = jnp.dot(h1, w12_ref[...], preferred_element_type=jnp.float32) \
        + b12_ref[...]
    h2 = _gelu(jnp.dot(h1, w21_ref[...], preferred_element_type=jnp.float32)
               + b21_ref[...])
    feats = jnp.dot(h2, w22_ref[...], preferred_element_type=jnp.float32) \
        + b22_ref[...]
    wt = jax.nn.relu(jnp.dot(feats, ww_ref[...],
                             preferred_element_type=jnp.float32) + wb_ref[...])
    alpha = wt / (jnp.sum(wt, axis=0, keepdims=True) + 1e-08)
    res = res_ref[...]
    colmask = jax.lax.broadcasted_iota(jnp.int32, res.shape, 1) != 0
    res = jnp.where(colmask, res, 0.0)
    pr = jnp.sum(res * alpha, axis=0, keepdims=True)
    out_ref[...] = pred0_ref[...] + pr


def _combine(xs, res, pred0, p):
    full = lambda a: pl.BlockSpec(a.shape, lambda *_: (0,) * a.ndim)
    args = [xs, res, pred0,
            p['m1_w1'].T, p['m1_b1'].reshape(1, 12),
            p['m1_w2'].T, p['m1_b2'].reshape(1, 8),
            p['m2_w1'].T, p['m2_b1'].reshape(1, 8),
            p['m2_w2'].T, p['m2_b2'].reshape(1, 8),
            p['wt_w'].T, p['wt_b'].reshape(1, 1)]
    return pl.pallas_call(
        _combine_body,
        in_specs=[full(a) for a in args],
        out_specs=pl.BlockSpec((1, 8), lambda *_: (0, 0)),
        out_shape=jax.ShapeDtypeStruct((1, 8), jnp.float32),
    )(*args)


def _hops_to_target0(src, dst, n):
    hop0 = jnp.full((n,), -1, dtype=jnp.int32).at[0].set(0)

    def cond(carry):
        hop, h = carry
        m = (hop[dst] == h) & (hop[src] == -1)
        return m.any()

    def body(carry):
        hop, h = carry
        m = (hop[dst] == h) & (hop[src] == -1)
        reached = jax.ops.segment_max(m.astype(jnp.int32), src,
                                      num_segments=n) > 0
        return jnp.where(reached, h + 1, hop), h + 1

    hop, _ = jax.lax.while_loop(cond, body, (hop0, jnp.int32(0)))
    return hop


def kernel(nodes, edge_index, edge_attr, valid, r, fx, loc, earray, params):
    p = params
    n = nodes.shape[0]
    res_idx = earray.shape[1]

    v = valid[0][:, :, None]
    nodes_m = jax.lax.stop_gradient(nodes * v)
    nodes_flat = nodes_m.reshape(n, -1)
    earray2 = jax.lax.stop_gradient(earray[0])
    vb = v != 0
    valid_mean = vb[:, :, 0].astype(jnp.float32).mean(axis=-1)

    src = edge_index[0].astype(jnp.int32)
    dst = edge_index[1].astype(jnp.int32)

    rf = _rain_mlp(r[0], p)

    logits = _edge_logits(edge_attr[:, :3], p) if edge_attr.shape[1] == 3 \
        else _edge_logits(edge_attr, p)

    hops = _hops_to_target0(src, dst, n)
    hs, hd = hops[src], hops[dst]
    on_path = (hs >= 0) & (hd >= 0) & (hs == hd + 1)
    max_hop = jnp.max(hops)

    neg_inf = jnp.float32(-jnp.inf)
    w = jnp.where(on_path, logits * valid_mean[src], neg_inf)
    m = jax.ops.segment_max(w, dst, num_segments=n)
    e = jnp.where(on_path, jnp.exp(w - m[dst]), 0.0)
    s = jax.ops.segment_sum(e, dst, num_segments=n)
    alpha = jnp.where(on_path, e / (s[dst] + 1e-16), 0.0)

    elev = jnp.where(on_path, hd + 1, 0)

    key = ((jnp.uint32(n + 1) - elev.astype(jnp.uint32)) << 16) \
        | dst.astype(jnp.uint32)
    order = jnp.argsort(key)

    CHK = 32768
    src_s = jnp.concatenate([src[order], jnp.zeros((CHK,), jnp.int32)])
    dst_s = jnp.concatenate([dst[order], jnp.full((CHK,), n, jnp.int32)])
    alpha_s = jnp.concatenate([alpha[order], jnp.zeros((CHK,), jnp.float32)])

    cnt = jnp.bincount(elev, length=n + 1)
    csum = jnp.cumsum(cnt)
    total = csum[n]

    def chunk_body(carry):
        x_flat, i, lo, hi = carry
        start = lo + i * CHK
        idx = start + jax.lax.iota(jnp.int32, CHK)
        ok = idx < hi
        sc = jax.lax.dynamic_slice(src_s, (start,), (CHK,))
        dc = jax.lax.dynamic_slice(dst_s, (start,), (CHK,))
        ac = jnp.where(ok, jax.lax.dynamic_slice(alpha_s, (start,), (CHK,)),
                       0.0)
        dc = jnp.where(ok, dc, n)
        msg = x_flat[sc] * ac[:, None]
        x_flat = x_flat.at[dc].add(msg, indices_are_sorted=True,
                                   unique_indices=False)
        return x_flat, i + 1, lo, hi

    def hop_cond(carry):
        _, h = carry
        return h >= 1

    def hop_body(carry):
        x_flat, h = carry
        lo = total - csum[h]
        hi = lo + cnt[h]

        def c_cond(carry):
            _, i, lo_, hi_ = carry
            return lo_ + i * CHK < hi_

        x_flat, _, _, _ = jax.lax.while_loop(
            c_cond, chunk_body, (x_flat, jnp.int32(0), lo, hi))
        return x_flat, h - 1

    x_flat0 = rf.reshape(n, 64)
    x_flat, _ = jax.lax.while_loop(hop_cond, hop_body, (x_flat0, max_hop))
    x_out = x_flat.reshape(n, 8, 8)

    B1 = res_idx
    B2 = res_idx + 1
    Bt = B1 + B2
    B_pad = ((Bt + 127) // 128) * 128
    batch = jnp.concatenate([
        rf[:B1],
        x_out[:B2],
        jnp.zeros((B_pad - Bt, 8, 8), jnp.float32),
    ], axis=0)
    head = _gru_head(batch, p)
    pred_coarse = head[:B1][:, :, None]
    pred_part = head[B1:B1 + B2]

    res = nodes_flat[1:res_idx + 1] - pred_part[1:]
    adjusted = _combine(earray2[:res_idx][:, :19], res, pred_part[:1], p)

    original_valid = vb[:, :, 0]
    return (adjusted, pred_part[:res_idx], pred_coarse, original_valid)


# speedup vs baseline: 2.8476x; 1.5204x over previous
import functools

import jax
import jax.numpy as jnp
from jax import lax
from jax.experimental import pallas as pl
from jax.experimental.pallas import tpu as pltpu, tpu_sc as plsc


def _gelu(x):
    return 0.5 * x * (1.0 + jax.lax.erf(x * 0.7071067811865476))


def _edge_logits_body(ea_ref, w1_ref, b1_ref, w2_ref, b2_ref, w3_ref, b3_ref,
                      out_ref):
    x = ea_ref[:, 0:3]
    h = _gelu(jnp.dot(x, w1_ref[...], preferred_element_type=jnp.float32)
              + b1_ref[...])
    h = _gelu(jnp.dot(h, w2_ref[...], preferred_element_type=jnp.float32)
              + b2_ref[...])
    out_ref[...] = (jnp.dot(h, w3_ref[...], preferred_element_type=jnp.float32)
                    + b3_ref[...])


def _edge_logits(edge_attr, p):
    E = edge_attr.shape[0]
    BLK = 8000
    grid = (E // BLK,)
    w1 = p['le_w1'].T
    b1 = p['le_b1'].reshape(1, 16)
    w2 = p['le_w2'].T
    b2 = p['le_b2'].reshape(1, 16)
    w3 = p['le_w3'].T
    b3 = p['le_b3'].reshape(1, 1)
    full = lambda a: pl.BlockSpec(a.shape, lambda i: (0,) * a.ndim)
    out = pl.pallas_call(
        _edge_logits_body,
        grid=grid,
        in_specs=[
            pl.BlockSpec((BLK, edge_attr.shape[1]), lambda i: (i, 0)),
            full(w1), full(b1), full(w2), full(b2), full(w3), full(b3),
        ],
        out_specs=pl.BlockSpec((BLK, 1), lambda i: (i, 0)),
        out_shape=jax.ShapeDtypeStruct((E, 1), jnp.float32),
    )(edge_attr, w1, b1, w2, b2, w3, b3)
    return out[:, 0]


def _rain_body(x_ref, w1_ref, b1_ref, w2_ref, b2_ref, out_ref):
    x = x_ref[...]
    h = _gelu(x * w1_ref[...] + b1_ref[...])
    out_ref[...] = (jnp.dot(h, w2_ref[...], preferred_element_type=jnp.float32)
                    + b2_ref[...])


def _rain_mlp(r0, p):
    N = r0.shape[0]
    x = r0.reshape(N * 8, 1)
    BLK = 8000
    grid = ((N * 8) // BLK,)
    w1 = p['rain_w1'].reshape(1, 8)
    b1 = p['rain_b1'].reshape(1, 8)
    w2 = p['rain_w2'].T
    b2 = p['rain_b2'].reshape(1, 8)
    full = lambda a: pl.BlockSpec(a.shape, lambda i: (0,) * a.ndim)
    out = pl.pallas_call(
        _rain_body,
        grid=grid,
        in_specs=[
            pl.BlockSpec((BLK, 1), lambda i: (i, 0)),
            full(w1), full(b1), full(w2), full(b2),
        ],
        out_specs=pl.BlockSpec((BLK, 8), lambda i: (i, 0)),
        out_shape=jax.ShapeDtypeStruct((N * 8, 8), jnp.float32),
    )(x, w1, b1, w2, b2)
    return out.reshape(N, 8, 8)


def _gru_head_body(x_ref, wih_ref, bih_ref, whh_ref, bhh_ref, pw_ref, pb_ref,
                   out_ref):
    B = x_ref.shape[0]
    h = jnp.zeros((B, 8), jnp.float32)
    cols = []
    for t in range(8):
        xt = x_ref[:, 8 * t:8 * t + 8]
        gi = jnp.dot(xt, wih_ref[...], preferred_element_type=jnp.float32) \
            + bih_ref[...]
        gh = jnp.dot(h, whh_ref[...], preferred_element_type=jnp.float32) \
            + bhh_ref[...]
        rg = jax.nn.sigmoid(gi[:, 0:8] + gh[:, 0:8])
        zg = jax.nn.sigmoid(gi[:, 8:16] + gh[:, 8:16])
        ng = jnp.tanh(gi[:, 16:24] + rg * gh[:, 16:24])
        h = (1.0 - zg) * ng + zg * h
        cols.append(jnp.dot(_gelu(h), pw_ref[...],
                            preferred_element_type=jnp.float32) + pb_ref[...])
    out_ref[...] = jnp.concatenate(cols, axis=1)


def _gru_head(x, p):
    B = x.shape[0]
    x2 = x.reshape(B, 64)
    BLK = 1040
    grid = (B // BLK,)
    wih = p['gru_wih'].T
    bih = p['gru_bih'].reshape(1, 24)
    whh = p['gru_whh'].T
    bhh = p['gru_bhh'].reshape(1, 24)
    pw = p['post_w'].T
    pb = p['post_b'].reshape(1, 1)
    full = lambda a: pl.BlockSpec(a.shape, lambda i: (0,) * a.ndim)
    return pl.pallas_call(
        _gru_head_body,
        grid=grid,
        in_specs=[pl.BlockSpec((BLK, 64), lambda i: (i, 0)),
                  full(wih), full(bih), full(whh), full(bhh),
                  full(pw), full(pb)],
        out_specs=pl.BlockSpec((BLK, 8), lambda i: (i, 0)),
        out_shape=jax.ShapeDtypeStruct((B, 8), jnp.float32),
    )(x2, wih, bih, whh, bhh, pw, pb)


def _combine_body(xs_ref, res_ref, pred0_ref, w11_ref, b11_ref, w12_ref,
                  b12_ref, w21_ref, b21_ref, w22_ref, b22_ref, ww_ref, wb_ref,
                  out_ref):
    xs = xs_ref[...]
    h1 = _gelu(jnp.dot(xs, w11_ref[...], preferred_element_type=jnp.float32)
               + b11_ref[...])
    h1 = jnp.dot(h1, w12_ref[...], preferred_element_type=jnp.float32) \
        + b12_ref[...]
    h2 = _gelu(jnp.dot(h1, w21_ref[...], preferred_element_type=jnp.float32)
               + b21_ref[...])
    feats = jnp.dot(h2, w22_ref[...], preferred_element_type=jnp.float32) \
        + b22_ref[...]
    wt = jax.nn.relu(jnp.dot(feats, ww_ref[...],
                             preferred_element_type=jnp.float32) + wb_ref[...])
    alpha = wt / (jnp.sum(wt, axis=0, keepdims=True) + 1e-08)
    res = res_ref[...]
    colmask = jax.lax.broadcasted_iota(jnp.int32, res.shape, 1) != 0
    res = jnp.where(colmask, res, 0.0)
    pr = jnp.sum(res * alpha, axis=0, keepdims=True)
    out_ref[...] = pred0_ref[...] + pr


def _combine(xs, res, pred0, p):
    full = lambda a: pl.BlockSpec(a.shape, lambda *_: (0,) * a.ndim)
    args = [xs, res, pred0,
            p['m1_w1'].T, p['m1_b1'].reshape(1, 12),
            p['m1_w2'].T, p['m1_b2'].reshape(1, 8),
            p['m2_w1'].T, p['m2_b1'].reshape(1, 8),
            p['m2_w2'].T, p['m2_b2'].reshape(1, 8),
            p['wt_w'].T, p['wt_b'].reshape(1, 1)]
    return pl.pallas_call(
        _combine_body,
        in_specs=[full(a) for a in args],
        out_specs=pl.BlockSpec((1, 8), lambda *_: (0, 0)),
        out_shape=jax.ShapeDtypeStruct((1, 8), jnp.float32),
    )(*args)


def _make_sc_gather(V, D, B):
    info = plsc.get_sparse_core_info()
    NW = info.num_cores * info.num_subcores
    assert D % info.num_lanes == 0 and B % (8 * NW) == 0
    b_per_w = B // NW
    mesh = plsc.VectorSubcoreMesh(core_axis_name="c", subcore_axis_name="s")

    @functools.partial(
        pl.kernel, mesh=mesh,
        out_type=jax.ShapeDtypeStruct((B, D), jnp.float32),
        scratch_types=[
            pltpu.VMEM((b_per_w,), jnp.int32),
            pltpu.VMEM((b_per_w, D), jnp.float32),
            pltpu.SemaphoreType.DMA,
        ],
    )
    def k(table_hbm, idx_hbm, out_hbm, idx_v, rows_v, sem):
        wid = lax.axis_index("s") * info.num_cores + lax.axis_index("c")
        base = wid * b_per_w
        pltpu.sync_copy(idx_hbm.at[pl.ds(base, b_per_w)], idx_v)
        pltpu.async_copy(table_hbm.at[idx_v], rows_v, sem).wait()
        pltpu.sync_copy(rows_v, out_hbm.at[pl.ds(base, b_per_w)])

    return k


def _hops_to_target0(src, dst, n):
    hop0 = jnp.full((n,), -1, dtype=jnp.int32).at[0].set(0)
    E = src.shape[0]
    both = jnp.concatenate([dst, src])

    def cond(carry):
        _, _, changed = carry
        return changed

    def body(carry):
        hop, h, _ = carry
        hb = hop[both]
        m = (hb[:E] == h) & (hb[E:] == -1)
        reached = jax.ops.segment_max(m.astype(jnp.int32), src,
                                      num_segments=n) > 0
        return jnp.where(reached, h + 1, hop), h + 1, m.any()

    hop, _, _ = jax.lax.while_loop(
        cond, body, (hop0, jnp.int32(0), jnp.bool_(True)))
    return hop


def kernel(nodes, edge_index, edge_attr, valid, r, fx, loc, earray, params):
    p = params
    n = nodes.shape[0]
    res_idx = earray.shape[1]

    v = valid[0][:, :, None]
    nodes_m = jax.lax.stop_gradient(nodes * v)
    nodes_flat = nodes_m.reshape(n, -1)
    earray2 = jax.lax.stop_gradient(earray[0])
    vb = v != 0
    valid_mean = vb[:, :, 0].astype(jnp.float32).mean(axis=-1)

    src = edge_index[0].astype(jnp.int32)
    dst = edge_index[1].astype(jnp.int32)

    rf = _rain_mlp(r[0], p)

    logits = _edge_logits(edge_attr[:, :3], p) if edge_attr.shape[1] == 3 \
        else _edge_logits(edge_attr, p)

    hops = _hops_to_target0(src, dst, n)
    E = src.shape[0]
    hb = hops[jnp.concatenate([src, dst])]
    hs, hd = hb[:E], hb[E:]
    on_path = (hs >= 0) & (hd >= 0) & (hs == hd + 1)
    max_hop = jnp.max(hops)

    neg_inf = jnp.float32(-jnp.inf)
    w = jnp.where(on_path, logits * valid_mean[src], neg_inf)
    m = jax.ops.segment_max(w, dst, num_segments=n)
    e = jnp.where(on_path, jnp.exp(w - m[dst]), 0.0)
    s = jax.ops.segment_sum(e, dst, num_segments=n)
    alpha = jnp.where(on_path, e / (s[dst] + 1e-16), 0.0)

    elev = jnp.where(on_path, hd + 1, 0)

    key = ((jnp.uint32(n + 1) - elev.astype(jnp.uint32)) << 16) \
        | dst.astype(jnp.uint32)
    order = jnp.argsort(key)

    CHK = 16384
    src_s = jnp.concatenate([src[order], jnp.zeros((CHK,), jnp.int32)])
    dst_s = jnp.concatenate([dst[order], jnp.full((CHK,), n, jnp.int32)])
    alpha_s = jnp.concatenate([alpha[order], jnp.zeros((CHK,), jnp.float32)])

    cnt = jnp.bincount(elev, length=n + 1)
    csum = jnp.cumsum(cnt)
    total = csum[n]

    sc_gather = _make_sc_gather(n, 128, CHK)

    def chunk_body(carry):
        x_flat, i, lo, hi = carry
        start = lo + i * CHK
        idx = start + jax.lax.iota(jnp.int32, CHK)
        ok = idx < hi
        sc = jax.lax.dynamic_slice(src_s, (start,), (CHK,))
        dc = jax.lax.dynamic_slice(dst_s, (start,), (CHK,))
        ac = jnp.where(ok, jax.lax.dynamic_slice(alpha_s, (start,), (CHK,)),
                       0.0)
        dc = jnp.where(ok, dc, n)
        msg = sc_gather(x_flat, sc) * ac[:, None]
        x_flat = x_flat.at[dc].add(msg, indices_are_sorted=True,
                                   unique_indices=False)
        return x_flat, i + 1, lo, hi

    def hop_cond(carry):
        _, h = carry
        return h >= 1

    def hop_body(carry):
        x_flat, h = carry
        lo = total - csum[h]
        hi = lo + cnt[h]

        def c_cond(carry):
            _, i, lo_, hi_ = carry
            return lo_ + i * CHK < hi_

        x_flat, _, _, _ = jax.lax.while_loop(
            c_cond, chunk_body, (x_flat, jnp.int32(0), lo, hi))
        return x_flat, h - 1

    x_flat0 = jnp.concatenate(
        [rf.reshape(n, 64), jnp.zeros((n, 64), jnp.float32)], axis=1)
    x_flat, _ = jax.lax.while_loop(hop_cond, hop_body, (x_flat0, max_hop))
    x_out = x_flat[:, :64].reshape(n, 8, 8)

    B1 = res_idx
    B2 = res_idx + 1
    Bt = B1 + B2
    B_pad = ((Bt + 127) // 128) * 128
    batch = jnp.concatenate([
        rf[:B1],
        x_out[:B2],
        jnp.zeros((B_pad - Bt, 8, 8), jnp.float32),
    ], axis=0)
    head = _gru_head(batch, p)
    pred_coarse = head[:B1][:, :, None]
    pred_part = head[B1:B1 + B2]

    res = nodes_flat[1:res_idx + 1] - pred_part[1:]
    adjusted = _combine(earray2[:res_idx][:, :19], res, pred_part[:1], p)

    original_valid = vb[:, :, 0]
    return (adjusted, pred_part[:res_idx], pred_coarse, original_valid)
